# baseline (speedup 1.0000x reference)
import jax
import jax.numpy as jnp
from jax.experimental import pallas as pl

N_NODES = 10000
HEADS = 8
HID = 16
D = HEADS * HID


def _edge_softmax(logits, dst, n):
    m = jax.ops.segment_max(logits, dst, num_segments=n)
    m = jnp.where(jnp.isfinite(m), m, 0.0)
    ex = jnp.exp(logits - m[dst])
    s = jax.ops.segment_sum(ex, dst, num_segments=n)
    return ex / (s[dst] + 1e-16)


def _gat_conv(x, src, dst, fc, attn_l, attn_r, heads, dout):
    n = x.shape[0]
    z = (x @ fc).reshape(n, heads, dout)
    el = jnp.sum(z * attn_l[None, :, :], axis=-1)
    er = jnp.sum(z * attn_r[None, :, :], axis=-1)
    logits = jax.nn.leaky_relu(el[src] + er[dst], negative_slope=0.2)
    alpha = _edge_softmax(logits, dst, n)
    msg = z[src] * alpha[:, :, None]
    out = jax.ops.segment_sum(msg, dst, num_segments=n)
    return out.reshape(n, heads * dout)


def _batchnorm(x, gamma, beta):
    mu = jnp.mean(x, axis=0, keepdims=True)
    var = jnp.var(x, axis=0, keepdims=True)
    return gamma * (x - mu) / jnp.sqrt(var + 1e-5) + beta


def _gat_layer(x, src, dst, p, heads, dout):
    h_in = x
    hh = _gat_conv(x, src, dst, p['fc'], p['attn_l'], p['attn_r'], heads, dout)
    hh = _batchnorm(hh, p['bn_g'], p['bn_b'])
    hh = jax.nn.elu(hh)
    if hh.shape[1] == h_in.shape[1]:
        hh = h_in + hh
    return hh


def _mlp_kernel(x_ref, w0_ref, b0_ref, w1_ref, b1_ref, w2_ref, b2_ref, y_ref):
    x = x_ref[...]
    y = jnp.maximum(x @ w0_ref[...] + b0_ref[...][None, :], 0.0)
    y = jnp.maximum(y @ w1_ref[...] + b1_ref[...][None, :], 0.0)
    y_ref[...] = y @ w2_ref[...] + b2_ref[...][None, :]


def kernel(h, edge_index, e, emb, params):
    src = edge_index[0]
    dst = edge_index[1]
    x = jnp.take(emb, h, axis=0)
    x = _gat_layer(x, src, dst, params['l0'], HEADS, HID)
    x = _gat_layer(x, src, dst, params['l1'], HEADS, HID)
    s = jax.nn.softmax(x @ params['l1']['assign_w'] + params['l1']['assign_b'], axis=-1)
    x = _gat_layer(x, src, dst, params['l2'], HEADS, HID)
    x = _gat_layer(x, src, dst, params['l3'], 1, HID)
    mlp = params['mlp']
    y = pl.pallas_call(
        _mlp_kernel,
        out_shape=jax.ShapeDtypeStruct((x.shape[0], mlp['w2'].shape[1]), jnp.float32),
    )(x, mlp['w0'], mlp['b0'], mlp['w1'], mlp['b1'], mlp['w2'], mlp['b2'])
    return (y, s)


# speedup vs baseline: 58.5592x; 58.5592x over previous
import functools

import jax
import jax.numpy as jnp
from jax import lax
from jax.experimental import pallas as pl
from jax.experimental.pallas import tpu as pltpu
from jax.experimental.pallas import tpu_sc as plsc

N = 10000
E = 320000
NPAD = 10240
HEADS = 8
HID = 16
D = HEADS * HID
NTILES = 32
EPW = E // NTILES
CHUNK = 80
ROWS_PER_TILE = NPAD // 16


def _make_edge_kernel(zw, heads):
    n_chunks = EPW // CHUNK
    mesh = plsc.VectorSubcoreMesh(core_axis_name="c", subcore_axis_name="s")

    @functools.partial(
        pl.kernel,
        mesh=mesh,
        compiler_params=pltpu.CompilerParams(use_tc_tiling_on_sc=False),
        out_type=jax.ShapeDtypeStruct((2, NPAD, zw), jnp.float32),
        scratch_types=[
            pltpu.VMEM((CHUNK, zw), jnp.float32),
            pltpu.VMEM((CHUNK, 16), jnp.float32),
            pltpu.VMEM((CHUNK,), jnp.int32),
            pltpu.VMEM((CHUNK,), jnp.int32),
            pltpu.VMEM((128, zw), jnp.float32),
            pltpu.VMEM_SHARED((NPAD, zw), jnp.float32),
            pltpu.SemaphoreType.DMA,
            pltpu.SemaphoreType.DMA,
        ],
    )
    def ek(zel_hbm, er_hbm, src_hbm, dst_hbm, out_hbm,
           zbuf, erbuf, srcb, dstb, zerob, acc, sem1, sem2):
        cid = lax.axis_index("c")
        sid = lax.axis_index("s")
        wid = sid * 2 + cid
        headmask = lax.iota(jnp.int32, 16) < heads

        def zrow(r, carry):
            for j in range(zw // 16):
                zerob[r, pl.ds(j * 16, 16)] = jnp.zeros((16,), jnp.float32)
            return carry
        lax.fori_loop(0, 128, zrow, 0)
        for b in range(ROWS_PER_TILE // 128):
            pltpu.sync_copy(zerob,
                            acc.at[pl.ds(sid * ROWS_PER_TILE + b * 128, 128)])
        plsc.subcore_barrier()

        def chunk(k, carry):
            base = wid * EPW + k * CHUNK
            pltpu.sync_copy(src_hbm.at[pl.ds(base, CHUNK)], srcb)
            pltpu.sync_copy(dst_hbm.at[pl.ds(base, CHUNK)], dstb)
            cp1 = pltpu.async_copy(zel_hbm.at[srcb], zbuf, sem1)
            cp2 = pltpu.async_copy(er_hbm.at[dstb], erbuf, sem2)
            cp1.wait()
            cp2.wait()

            def edge(c, carry2):
                elv = zbuf[c, pl.ds(zw - 16, 16)]
                erv = erbuf[c, pl.ds(0, 16)]
                t = elv + erv
                t = jnp.maximum(t, 0.2 * t)
                exv = jnp.exp(t)
                exv = jnp.where(headmask, exv, 0.0)
                zbuf[c, pl.ds(zw - 16, 16)] = exv
                for h in range(heads):
                    s = exv[h]
                    zbuf[c, pl.ds(h * 16, 16)] = zbuf[c, pl.ds(h * 16, 16)] * s
                return carry2
            lax.fori_loop(0, CHUNK, edge, 0)
            pltpu.sync_copy(zbuf, acc.at[dstb], add=True)
            return carry
        lax.fori_loop(0, n_chunks, chunk, 0)
        plsc.subcore_barrier()
        for b in range(ROWS_PER_TILE // 128):
            r0 = sid * ROWS_PER_TILE + b * 128
            pltpu.sync_copy(acc.at[pl.ds(r0, 128)],
                            out_hbm.at[cid, pl.ds(r0, 128)])

    return ek


_edge8 = _make_edge_kernel(144, 8)
_edge1 = _make_edge_kernel(32, 1)


def _k0_body(h_ref, emb_ref, x_ref):
    iot = lax.broadcasted_iota(jnp.int32, (N, 32), 1)
    oh = (h_ref[...] == iot).astype(jnp.float32)
    x_ref[...] = jnp.dot(oh, emb_ref[...], preferred_element_type=jnp.float32, precision=lax.Precision.HIGHEST)


def _post_body(acc_ref, xin_ref, b8_ref, g_ref, beta_ref, outx_ref):
    a = acc_ref[0, :N, :] + acc_ref[1, :N, :]
    den = jnp.dot(a[:, 128:136], b8_ref[...], preferred_element_type=jnp.float32, precision=lax.Precision.HIGHEST)
    hh = a[:, 0:128] / (den + 1e-16)
    mu = jnp.mean(hh, axis=0, keepdims=True)
    var = jnp.mean((hh - mu) ** 2, axis=0, keepdims=True)
    hh = g_ref[...][None, :] * (hh - mu) / jnp.sqrt(var + 1e-5) + beta_ref[...][None, :]
    hh = jnp.where(hh > 0, hh, jnp.exp(hh) - 1.0)
    outx_ref[...] = xin_ref[...] + hh


def _pre_body(x_ref, fc_ref, alp_ref, arp_ref, *rest):
    if len(rest) == 2:
        (zel_ref, er_ref), assign = rest, ()
    else:
        aw_ref, ab_ref, zel_ref, er_ref, s_ref = rest
        assign = (aw_ref, ab_ref, s_ref)
    x = x_ref[...]
    z = jnp.dot(x, fc_ref[...], preferred_element_type=jnp.float32,
                precision=lax.Precision.DEFAULT)
    zd = z.shape[1]
    zel_ref[:, pl.ds(0, zd)] = z
    zel_ref[:, pl.ds(zd, 16)] = jnp.dot(
        z, alp_ref[...], preferred_element_type=jnp.float32,
        precision=lax.Precision.HIGHEST)
    er_ref[...] = jnp.dot(z, arp_ref[...], preferred_element_type=jnp.float32,
                          precision=lax.Precision.HIGHEST)
    if assign:
        aw_ref, ab_ref, s_ref = assign
        logits = jnp.dot(x, aw_ref[...], preferred_element_type=jnp.float32,
                         precision=lax.Precision.DEFAULT)
        logits = logits + ab_ref[...][None, :]
        m = jnp.max(logits, axis=1, keepdims=True)
        ex = jnp.exp(logits - m)
        s_ref[...] = ex / jnp.sum(ex, axis=1, keepdims=True)


def _k4_body(acc_ref, bsel_ref, g_ref, beta_ref,
             w0_ref, b0_ref, w1_ref, b1_ref, w2_ref, b2_ref, y_ref):
    a = acc_ref[0, :N, :] + acc_ref[1, :N, :]
    den = jnp.dot(a[:, 16:32], bsel_ref[...], preferred_element_type=jnp.float32, precision=lax.Precision.HIGHEST)
    hh = a[:, 0:16] / (den + 1e-16)
    mu = jnp.mean(hh, axis=0, keepdims=True)
    var = jnp.mean((hh - mu) ** 2, axis=0, keepdims=True)
    hh = g_ref[...][None, :] * (hh - mu) / jnp.sqrt(var + 1e-5) + beta_ref[...][None, :]
    hh = jnp.where(hh > 0, hh, jnp.exp(hh) - 1.0)
    y = jnp.maximum(jnp.dot(hh, w0_ref[...], preferred_element_type=jnp.float32, precision=lax.Precision.DEFAULT)
                    + b0_ref[...][None, :], 0.0)
    y = jnp.maximum(jnp.dot(y, w1_ref[...], preferred_element_type=jnp.float32, precision=lax.Precision.DEFAULT)
                    + b1_ref[...][None, :], 0.0)
    y_ref[...] = jnp.dot(y, w2_ref[...], preferred_element_type=jnp.float32, precision=lax.Precision.DEFAULT) \
        + b2_ref[...][None, :]


def _attn_proj(p, heads, dout):
    alw = (p['attn_l'][:, :, None] * jnp.eye(heads)[:, None, :]).reshape(heads * dout, heads)
    arw = (p['attn_r'][:, :, None] * jnp.eye(heads)[:, None, :]).reshape(heads * dout, heads)
    pad = jnp.zeros((heads * dout, 16 - heads), jnp.float32)
    return jnp.concatenate([alw, pad], axis=1), jnp.concatenate([arw, pad], axis=1)


def kernel(h, edge_index, e, emb, params):
    with jax.default_matmul_precision("highest"):
        return _kernel_impl(h, edge_index, e, emb, params)


def _kernel_impl(h, edge_index, e, emb, params):
    src = edge_index[0]
    dst = edge_index[1]
    p0, p1, p2, p3 = params['l0'], params['l1'], params['l2'], params['l3']
    mlp = params['mlp']

    b8 = (lax.broadcasted_iota(jnp.int32, (8, 128), 1) // 16
          == lax.broadcasted_iota(jnp.int32, (8, 128), 0)).astype(jnp.float32)
    bsel = (lax.broadcasted_iota(jnp.int32, (16, 16), 0) == 0).astype(jnp.float32)

    x0 = pl.pallas_call(
        _k0_body,
        out_shape=jax.ShapeDtypeStruct((N, 128), jnp.float32),
    )(h.reshape(N, 1).astype(jnp.int32), emb)

    def pre(x, p, heads, aw=None, ab=None):
        alp, arp = _attn_proj(p, heads, HID)
        zd = p['fc'].shape[1]
        outs = [jax.ShapeDtypeStruct((N, zd + 16), jnp.float32),
                jax.ShapeDtypeStruct((N, 16), jnp.float32)]
        args = [x, p['fc'], alp, arp]
        if aw is not None:
            outs.append(jax.ShapeDtypeStruct((N, 100), jnp.float32))
            args += [aw, ab]
        return pl.pallas_call(_pre_body, out_shape=tuple(outs))(*args)

    def post(acc, x, p):
        return pl.pallas_call(
            _post_body, out_shape=jax.ShapeDtypeStruct((N, 128), jnp.float32),
        )(acc, x, b8, p['bn_g'], p['bn_b'])

    ze0, er0 = pre(x0, p0, HEADS)
    acc0 = _edge8(ze0, er0, src, dst)
    x1 = post(acc0, x0, p0)
    ze1, er1 = pre(x1, p1, HEADS)
    acc1 = _edge8(ze1, er1, src, dst)
    x2 = post(acc1, x1, p1)
    ze2, er2, s = pre(x2, p2, HEADS, p1['assign_w'], p1['assign_b'])
    acc2 = _edge8(ze2, er2, src, dst)
    x3 = post(acc2, x2, p2)
    ze3, er3 = pre(x3, p3, 1)
    acc3 = _edge1(ze3, er3, src, dst)
    y = pl.pallas_call(
        _k4_body,
        out_shape=jax.ShapeDtypeStruct((N, 6), jnp.float32),
    )(acc3, bsel, p3['bn_g'], p3['bn_b'],
      mlp['w0'], mlp['b0'], mlp['w1'], mlp['b1'], mlp['w2'], mlp['b2'])
    return (y, s)


# speedup vs baseline: 83.8607x; 1.4321x over previous
import functools

import jax
import jax.numpy as jnp
from jax import lax
from jax.experimental import pallas as pl
from jax.experimental.pallas import tpu as pltpu
from jax.experimental.pallas import tpu_sc as plsc

N = 10000
E = 320000
NPAD = 10240
HEADS = 8
HID = 16
D = HEADS * HID
NTILES = 32
EPW = E // NTILES
CHUNK = 40
ROWS_PER_TILE = NPAD // 16
ZR = 32


UNROLL = 5
SPAN = UNROLL * CHUNK
N_ITERS = EPW // SPAN


def _make_edge_kernel(zw, heads):
    mesh = plsc.VectorSubcoreMesh(core_axis_name="c", subcore_axis_name="s")

    @functools.partial(
        pl.kernel,
        mesh=mesh,
        compiler_params=pltpu.CompilerParams(use_tc_tiling_on_sc=False),
        out_type=jax.ShapeDtypeStruct((2, NPAD, zw), jnp.float32),
        scratch_types=[
            pltpu.VMEM((UNROLL, CHUNK, zw), jnp.float32),
            pltpu.VMEM((UNROLL, CHUNK, 16), jnp.float32),
            pltpu.VMEM((SPAN,), jnp.int32),
            pltpu.VMEM((UNROLL, CHUNK), jnp.int32),
            pltpu.VMEM((ZR, zw), jnp.float32),
            pltpu.VMEM_SHARED((NPAD, zw), jnp.float32),
            pltpu.SemaphoreType.DMA,
        ] + [pltpu.SemaphoreType.DMA] * (2 * UNROLL),
    )
    def ek(src_hbm, dst_hbm, zel_hbm, er_hbm, out_hbm,
           zbuf, erbuf, srcb, dstb, zerob, acc, isem, *sems):
        gsem = sems[:UNROLL]
        ssem = sems[UNROLL:]
        cid = lax.axis_index("c")
        sid = lax.axis_index("s")
        wid = sid * 2 + cid

        def zrow(r, carry):
            for j in range(zw // 16):
                zerob[r, pl.ds(j * 16, 16)] = jnp.zeros((16,), jnp.float32)
            return carry
        lax.fori_loop(0, ZR, zrow, 0)

        def zcopy(b, carry):
            pltpu.sync_copy(zerob,
                            acc.at[pl.ds(sid * ROWS_PER_TILE + b * ZR, ZR)])
            return carry
        lax.fori_loop(0, ROWS_PER_TILE // ZR, zcopy, 0)
        plsc.subcore_barrier()

        def span_iter(k, carry):
            base = wid * EPW + k * SPAN
            icps = [pltpu.async_copy(src_hbm.at[pl.ds(base, SPAN)], srcb, isem)]
            for u in range(UNROLL):
                icps.append(pltpu.async_copy(
                    dst_hbm.at[pl.ds(base + u * CHUNK, CHUNK)],
                    dstb.at[u], isem))
            for cp in icps:
                cp.wait()
            gcps = []
            for u in range(UNROLL):
                gcps.append((
                    pltpu.async_copy(
                        zel_hbm.at[srcb.at[pl.ds(u * CHUNK, CHUNK)]],
                        zbuf.at[u], gsem[u]),
                    pltpu.async_copy(er_hbm.at[dstb.at[u]],
                                     erbuf.at[u], gsem[u])))
            scps = []
            for u in range(UNROLL):
                gcps[u][0].wait()
                gcps[u][1].wait()

                def edge(c, carry2, u=u):
                    elv = zbuf[u, c, pl.ds(zw - 16, 16)]
                    erv = erbuf[u, c, pl.ds(0, 16)]
                    t = elv + erv
                    t = jnp.maximum(t, 0.2 * t)
                    exv = jnp.exp(t)
                    zbuf[u, c, pl.ds(zw - 16, 16)] = exv
                    for hd in range(heads):
                        s = exv[hd]
                        zbuf[u, c, pl.ds(hd * 16, 16)] = \
                            zbuf[u, c, pl.ds(hd * 16, 16)] * s
                    return carry2
                lax.fori_loop(0, CHUNK, edge, 0)
                scps.append(pltpu.async_copy(
                    zbuf.at[u], acc.at[dstb.at[u]], ssem[u], add=True))
            for cp in scps:
                cp.wait()
            return carry
        lax.fori_loop(0, N_ITERS, span_iter, 0)
        plsc.subcore_barrier()
        for b in range(ROWS_PER_TILE // 128):
            r0 = sid * ROWS_PER_TILE + b * 128
            pltpu.sync_copy(acc.at[pl.ds(r0, 128)],
                            out_hbm.at[cid, pl.ds(r0, 128)])

    return ek


_edge8 = _make_edge_kernel(144, 8)
_edge1 = _make_edge_kernel(32, 1)


def _k0_body(h_ref, emb_ref, x_ref):
    iot = lax.broadcasted_iota(jnp.int32, (N, 32), 1)
    oh = (h_ref[...] == iot).astype(jnp.float32)
    x_ref[...] = jnp.dot(oh, emb_ref[...], preferred_element_type=jnp.float32, precision=lax.Precision.HIGHEST)


def _post_body(acc_ref, xin_ref, b8_ref, g_ref, beta_ref, outx_ref):
    a = acc_ref[0, :N, :] + acc_ref[1, :N, :]
    den = jnp.dot(a[:, 128:136], b8_ref[...], preferred_element_type=jnp.float32, precision=lax.Precision.HIGHEST)
    hh = a[:, 0:128] / (den + 1e-16)
    mu = jnp.mean(hh, axis=0, keepdims=True)
    var = jnp.mean((hh - mu) ** 2, axis=0, keepdims=True)
    hh = g_ref[...][None, :] * (hh - mu) / jnp.sqrt(var + 1e-5) + beta_ref[...][None, :]
    hh = jnp.where(hh > 0, hh, jnp.exp(hh) - 1.0)
    outx_ref[...] = xin_ref[...] + hh


def _pre_body(x_ref, fc_ref, alp_ref, arp_ref, *rest):
    if len(rest) == 2:
        (zel_ref, er_ref), assign = rest, ()
    else:
        aw_ref, ab_ref, zel_ref, er_ref, s_ref = rest
        assign = (aw_ref, ab_ref, s_ref)
    x = x_ref[...]
    z = jnp.dot(x, fc_ref[...], preferred_element_type=jnp.float32,
                precision=lax.Precision.DEFAULT)
    zd = z.shape[1]
    zel_ref[:, pl.ds(0, zd)] = z
    zel_ref[:, pl.ds(zd, 16)] = jnp.dot(
        z, alp_ref[...], preferred_element_type=jnp.float32,
        precision=lax.Precision.HIGHEST)
    er_ref[...] = jnp.dot(z, arp_ref[...], preferred_element_type=jnp.float32,
                          precision=lax.Precision.HIGHEST)
    if assign:
        aw_ref, ab_ref, s_ref = assign
        logits = jnp.dot(x, aw_ref[...], preferred_element_type=jnp.float32,
                         precision=lax.Precision.DEFAULT)
        logits = logits + ab_ref[...][None, :]
        m = jnp.max(logits, axis=1, keepdims=True)
        ex = jnp.exp(logits - m)
        s_ref[...] = ex / jnp.sum(ex, axis=1, keepdims=True)


def _k4_body(acc_ref, bsel_ref, g_ref, beta_ref,
             w0_ref, b0_ref, w1_ref, b1_ref, w2_ref, b2_ref, y_ref):
    a = acc_ref[0, :N, :] + acc_ref[1, :N, :]
    den = jnp.dot(a[:, 16:32], bsel_ref[...], preferred_element_type=jnp.float32, precision=lax.Precision.HIGHEST)
    hh = a[:, 0:16] / (den + 1e-16)
    mu = jnp.mean(hh, axis=0, keepdims=True)
    var = jnp.mean((hh - mu) ** 2, axis=0, keepdims=True)
    hh = g_ref[...][None, :] * (hh - mu) / jnp.sqrt(var + 1e-5) + beta_ref[...][None, :]
    hh = jnp.where(hh > 0, hh, jnp.exp(hh) - 1.0)
    y = jnp.maximum(jnp.dot(hh, w0_ref[...], preferred_element_type=jnp.float32, precision=lax.Precision.DEFAULT)
                    + b0_ref[...][None, :], 0.0)
    y = jnp.maximum(jnp.dot(y, w1_ref[...], preferred_element_type=jnp.float32, precision=lax.Precision.DEFAULT)
                    + b1_ref[...][None, :], 0.0)
    y_ref[...] = jnp.dot(y, w2_ref[...], preferred_element_type=jnp.float32, precision=lax.Precision.DEFAULT) \
        + b2_ref[...][None, :]


def _attn_proj(p, heads, dout):
    alw = (p['attn_l'][:, :, None] * jnp.eye(heads)[:, None, :]).reshape(heads * dout, heads)
    arw = (p['attn_r'][:, :, None] * jnp.eye(heads)[:, None, :]).reshape(heads * dout, heads)
    pad = jnp.zeros((heads * dout, 16 - heads), jnp.float32)
    return jnp.concatenate([alw, pad], axis=1), jnp.concatenate([arw, pad], axis=1)


def kernel(h, edge_index, e, emb, params):
    with jax.default_matmul_precision("highest"):
        return _kernel_impl(h, edge_index, e, emb, params)


def _kernel_impl(h, edge_index, e, emb, params):
    src = edge_index[0]
    dst = edge_index[1]
    p0, p1, p2, p3 = params['l0'], params['l1'], params['l2'], params['l3']
    mlp = params['mlp']

    b8 = (lax.broadcasted_iota(jnp.int32, (8, 128), 1) // 16
          == lax.broadcasted_iota(jnp.int32, (8, 128), 0)).astype(jnp.float32)
    bsel = (lax.broadcasted_iota(jnp.int32, (16, 16), 0) == 0).astype(jnp.float32)

    x0 = pl.pallas_call(
        _k0_body,
        out_shape=jax.ShapeDtypeStruct((N, 128), jnp.float32),
    )(h.reshape(N, 1).astype(jnp.int32), emb)

    def pre(x, p, heads, aw=None, ab=None):
        alp, arp = _attn_proj(p, heads, HID)
        zd = p['fc'].shape[1]
        outs = [jax.ShapeDtypeStruct((N, zd + 16), jnp.float32),
                jax.ShapeDtypeStruct((N, 16), jnp.float32)]
        args = [x, p['fc'], alp, arp]
        if aw is not None:
            outs.append(jax.ShapeDtypeStruct((N, 100), jnp.float32))
            args += [aw, ab]
        return pl.pallas_call(_pre_body, out_shape=tuple(outs))(*args)

    def post(acc, x, p):
        return pl.pallas_call(
            _post_body, out_shape=jax.ShapeDtypeStruct((N, 128), jnp.float32),
        )(acc, x, b8, p['bn_g'], p['bn_b'])

    ze0, er0 = pre(x0, p0, HEADS)
    acc0 = _edge8(src, dst, ze0, er0)
    x1 = post(acc0, x0, p0)
    ze1, er1 = pre(x1, p1, HEADS)
    acc1 = _edge8(src, dst, ze1, er1)
    x2 = post(acc1, x1, p1)
    ze2, er2, s = pre(x2, p2, HEADS, p1['assign_w'], p1['assign_b'])
    acc2 = _edge8(src, dst, ze2, er2)
    x3 = post(acc2, x2, p2)
    ze3, er3 = pre(x3, p3, 1)
    acc3 = _edge1(src, dst, ze3, er3)
    y = pl.pallas_call(
        _k4_body,
        out_shape=jax.ShapeDtypeStruct((N, 6), jnp.float32),
    )(acc3, bsel, p3['bn_g'], p3['bn_b'],
      mlp['w0'], mlp['b0'], mlp['w1'], mlp['b1'], mlp['w2'], mlp['b2'])
    return (y, s)


# speedup vs baseline: 88.5634x; 1.0561x over previous
import functools

import jax
import jax.numpy as jnp
from jax import lax
from jax.experimental import pallas as pl
from jax.experimental.pallas import tpu as pltpu
from jax.experimental.pallas import tpu_sc as plsc

N = 10000
E = 320000
NPAD = 10240
HEADS = 8
HID = 16
D = HEADS * HID
NTILES = 32
EPW = E // NTILES
CHUNK = 40
ROWS_PER_TILE = NPAD // 16
ZR = 32


UNROLL = 5
SPAN = UNROLL * CHUNK
N_ITERS = EPW // SPAN


def _make_edge_kernel(zw, heads):
    mesh = plsc.VectorSubcoreMesh(core_axis_name="c", subcore_axis_name="s")

    @functools.partial(
        pl.kernel,
        mesh=mesh,
        compiler_params=pltpu.CompilerParams(use_tc_tiling_on_sc=False),
        out_type=jax.ShapeDtypeStruct((2, NPAD, zw), jnp.float32),
        scratch_types=[
            pltpu.VMEM((UNROLL, CHUNK, zw), jnp.float32),
            pltpu.VMEM((UNROLL, CHUNK, 16), jnp.float32),
            pltpu.VMEM((2, SPAN), jnp.int32),
            pltpu.VMEM((2 * UNROLL, CHUNK), jnp.int32),
            pltpu.VMEM((ZR, zw), jnp.float32),
            pltpu.VMEM_SHARED((NPAD, zw), jnp.float32),
            pltpu.SemaphoreType.DMA,
        ] + [pltpu.SemaphoreType.DMA] * (2 * UNROLL),
    )
    def ek(src_hbm, dst_hbm, zel_hbm, er_hbm, out_hbm,
           zbuf, erbuf, srcb, dstb, zerob, acc, isem, *sems):
        gsem = sems[:UNROLL]
        ssem = sems[UNROLL:]
        cid = lax.axis_index("c")
        sid = lax.axis_index("s")
        wid = sid * 2 + cid

        def zrow(r, carry):
            for j in range(zw // 16):
                zerob[r, pl.ds(j * 16, 16)] = jnp.zeros((16,), jnp.float32)
            return carry
        lax.fori_loop(0, ZR, zrow, 0)

        def zcopy(b, carry):
            pltpu.sync_copy(zerob,
                            acc.at[pl.ds(sid * ROWS_PER_TILE + b * ZR, ZR)])
            return carry
        lax.fori_loop(0, ROWS_PER_TILE // ZR, zcopy, 0)
        plsc.subcore_barrier()

        def issue_idx(k):
            b = lax.rem(k, 2)
            base = lax.min(wid * EPW + k * SPAN, E - SPAN)
            pltpu.async_copy(src_hbm.at[pl.ds(base, SPAN)], srcb.at[b], isem)
            for u in range(UNROLL):
                pltpu.async_copy(dst_hbm.at[pl.ds(base + u * CHUNK, CHUNK)],
                                 dstb.at[b * UNROLL + u], isem)

        def drain_idx():
            pltpu.make_async_copy(
                src_hbm.at[pl.ds(0, SPAN)], srcb.at[0], isem).wait()
            for u in range(UNROLL):
                pltpu.make_async_copy(
                    dst_hbm.at[pl.ds(0, CHUNK)], dstb.at[u], isem).wait()

        issue_idx(0)

        def span_iter(k, carry):
            b = lax.rem(k, 2)
            issue_idx(k + 1)
            drain_idx()
            gcps = []
            for u in range(UNROLL):
                gcps.append((
                    pltpu.async_copy(
                        zel_hbm.at[srcb.at[b, pl.ds(u * CHUNK, CHUNK)]],
                        zbuf.at[u], gsem[u]),
                    pltpu.async_copy(er_hbm.at[dstb.at[b * UNROLL + u]],
                                     erbuf.at[u], gsem[u])))
            scps = []
            for u in range(UNROLL):
                gcps[u][0].wait()
                gcps[u][1].wait()

                def edge(c, carry2, u=u):
                    elv = zbuf[u, c, pl.ds(zw - 16, 16)]
                    erv = erbuf[u, c, pl.ds(0, 16)]
                    t = elv + erv
                    t = jnp.maximum(t, 0.2 * t)
                    exv = jnp.exp(t)
                    zbuf[u, c, pl.ds(zw - 16, 16)] = exv
                    for hd in range(heads):
                        s = exv[hd]
                        zbuf[u, c, pl.ds(hd * 16, 16)] = \
                            zbuf[u, c, pl.ds(hd * 16, 16)] * s
                    return carry2
                lax.fori_loop(0, CHUNK, edge, 0, unroll=4)
                scps.append(pltpu.async_copy(
                    zbuf.at[u], acc.at[dstb.at[b * UNROLL + u]],
                    ssem[u], add=True))
            for cp in scps:
                cp.wait()
            return carry
        lax.fori_loop(0, N_ITERS, span_iter, 0)
        drain_idx()
        plsc.subcore_barrier()
        for b in range(ROWS_PER_TILE // 128):
            r0 = sid * ROWS_PER_TILE + b * 128
            pltpu.sync_copy(acc.at[pl.ds(r0, 128)],
                            out_hbm.at[cid, pl.ds(r0, 128)])

    return ek


_edge8 = _make_edge_kernel(144, 8)
_edge1 = _make_edge_kernel(32, 1)


def _k0_body(h_ref, emb_ref, x_ref):
    iot = lax.broadcasted_iota(jnp.int32, (N, 32), 1)
    oh = (h_ref[...] == iot).astype(jnp.float32)
    x_ref[...] = jnp.dot(oh, emb_ref[...], preferred_element_type=jnp.float32, precision=lax.Precision.HIGHEST)


def _post_body(acc_ref, xin_ref, b8_ref, g_ref, beta_ref, outx_ref):
    a = acc_ref[0, :N, :] + acc_ref[1, :N, :]
    den = jnp.dot(a[:, 128:136], b8_ref[...], preferred_element_type=jnp.float32, precision=lax.Precision.HIGHEST)
    hh = a[:, 0:128] / (den + 1e-16)
    mu = jnp.mean(hh, axis=0, keepdims=True)
    var = jnp.mean((hh - mu) ** 2, axis=0, keepdims=True)
    hh = g_ref[...][None, :] * (hh - mu) / jnp.sqrt(var + 1e-5) + beta_ref[...][None, :]
    hh = jnp.where(hh > 0, hh, jnp.exp(hh) - 1.0)
    outx_ref[...] = xin_ref[...] + hh


def _pre_body(x_ref, fc_ref, alp_ref, arp_ref, *rest):
    if len(rest) == 2:
        (zel_ref, er_ref), assign = rest, ()
    else:
        aw_ref, ab_ref, zel_ref, er_ref, s_ref = rest
        assign = (aw_ref, ab_ref, s_ref)
    x = x_ref[...]
    z = jnp.dot(x, fc_ref[...], preferred_element_type=jnp.float32,
                precision=lax.Precision.DEFAULT)
    zd = z.shape[1]
    zel_ref[:, pl.ds(0, zd)] = z
    zel_ref[:, pl.ds(zd, 16)] = jnp.dot(
        z, alp_ref[...], preferred_element_type=jnp.float32,
        precision=lax.Precision.HIGHEST)
    er_ref[...] = jnp.dot(z, arp_ref[...], preferred_element_type=jnp.float32,
                          precision=lax.Precision.HIGHEST)
    if assign:
        aw_ref, ab_ref, s_ref = assign
        logits = jnp.dot(x, aw_ref[...], preferred_element_type=jnp.float32,
                         precision=lax.Precision.DEFAULT)
        logits = logits + ab_ref[...][None, :]
        m = jnp.max(logits, axis=1, keepdims=True)
        ex = jnp.exp(logits - m)
        s_ref[...] = ex / jnp.sum(ex, axis=1, keepdims=True)


def _k4_body(acc_ref, bsel_ref, g_ref, beta_ref,
             w0_ref, b0_ref, w1_ref, b1_ref, w2_ref, b2_ref, y_ref):
    a = acc_ref[0, :N, :] + acc_ref[1, :N, :]
    den = jnp.dot(a[:, 16:32], bsel_ref[...], preferred_element_type=jnp.float32, precision=lax.Precision.HIGHEST)
    hh = a[:, 0:16] / (den + 1e-16)
    mu = jnp.mean(hh, axis=0, keepdims=True)
    var = jnp.mean((hh - mu) ** 2, axis=0, keepdims=True)
    hh = g_ref[...][None, :] * (hh - mu) / jnp.sqrt(var + 1e-5) + beta_ref[...][None, :]
    hh = jnp.where(hh > 0, hh, jnp.exp(hh) - 1.0)
    y = jnp.maximum(jnp.dot(hh, w0_ref[...], preferred_element_type=jnp.float32, precision=lax.Precision.DEFAULT)
                    + b0_ref[...][None, :], 0.0)
    y = jnp.maximum(jnp.dot(y, w1_ref[...], preferred_element_type=jnp.float32, precision=lax.Precision.DEFAULT)
                    + b1_ref[...][None, :], 0.0)
    y_ref[...] = jnp.dot(y, w2_ref[...], preferred_element_type=jnp.float32, precision=lax.Precision.DEFAULT) \
        + b2_ref[...][None, :]


def _attn_proj(p, heads, dout):
    alw = (p['attn_l'][:, :, None] * jnp.eye(heads)[:, None, :]).reshape(heads * dout, heads)
    arw = (p['attn_r'][:, :, None] * jnp.eye(heads)[:, None, :]).reshape(heads * dout, heads)
    pad = jnp.zeros((heads * dout, 16 - heads), jnp.float32)
    return jnp.concatenate([alw, pad], axis=1), jnp.concatenate([arw, pad], axis=1)


def kernel(h, edge_index, e, emb, params):
    with jax.default_matmul_precision("highest"):
        return _kernel_impl(h, edge_index, e, emb, params)


def _kernel_impl(h, edge_index, e, emb, params):
    src = edge_index[0]
    dst = edge_index[1]
    p0, p1, p2, p3 = params['l0'], params['l1'], params['l2'], params['l3']
    mlp = params['mlp']

    b8 = (lax.broadcasted_iota(jnp.int32, (8, 128), 1) // 16
          == lax.broadcasted_iota(jnp.int32, (8, 128), 0)).astype(jnp.float32)
    bsel = (lax.broadcasted_iota(jnp.int32, (16, 16), 0) == 0).astype(jnp.float32)

    x0 = pl.pallas_call(
        _k0_body,
        out_shape=jax.ShapeDtypeStruct((N, 128), jnp.float32),
    )(h.reshape(N, 1).astype(jnp.int32), emb)

    def pre(x, p, heads, aw=None, ab=None):
        alp, arp = _attn_proj(p, heads, HID)
        zd = p['fc'].shape[1]
        outs = [jax.ShapeDtypeStruct((N, zd + 16), jnp.float32),
                jax.ShapeDtypeStruct((N, 16), jnp.float32)]
        args = [x, p['fc'], alp, arp]
        if aw is not None:
            outs.append(jax.ShapeDtypeStruct((N, 100), jnp.float32))
            args += [aw, ab]
        return pl.pallas_call(_pre_body, out_shape=tuple(outs))(*args)

    def post(acc, x, p):
        return pl.pallas_call(
            _post_body, out_shape=jax.ShapeDtypeStruct((N, 128), jnp.float32),
        )(acc, x, b8, p['bn_g'], p['bn_b'])

    ze0, er0 = pre(x0, p0, HEADS)
    acc0 = _edge8(src, dst, ze0, er0)
    x1 = post(acc0, x0, p0)
    ze1, er1 = pre(x1, p1, HEADS)
    acc1 = _edge8(src, dst, ze1, er1)
    x2 = post(acc1, x1, p1)
    ze2, er2, s = pre(x2, p2, HEADS, p1['assign_w'], p1['assign_b'])
    acc2 = _edge8(src, dst, ze2, er2)
    x3 = post(acc2, x2, p2)
    ze3, er3 = pre(x3, p3, 1)
    acc3 = _edge1(src, dst, ze3, er3)
    y = pl.pallas_call(
        _k4_body,
        out_shape=jax.ShapeDtypeStruct((N, 6), jnp.float32),
    )(acc3, bsel, p3['bn_g'], p3['bn_b'],
      mlp['w0'], mlp['b0'], mlp['w1'], mlp['b1'], mlp['w2'], mlp['b2'])
    return (y, s)
